# baseline (speedup 1.0000x reference)
import functools

import jax
import jax.numpy as jnp
from jax import lax
from jax.experimental import pallas as pl
from jax.experimental.pallas import tpu as pltpu
from jax.experimental.pallas import tpu_sc as plsc

E_DIM = 32
BATCH = 16384
NEGB = 32768
NW = 32
P = BATCH // NW
N = NEGB // NW
PC = P // 128
NC = N // 128
G16 = 16

_mesh = plsc.VectorSubcoreMesh(core_axis_name="c", subcore_axis_name="s")


def _sc_scores(fs_hbm, fp_hbm, fo_hbm, no1_hbm, ns2_hbm, etab, rtab,
               true_hbm, f1_hbm, f2_hbm,
               sidx, pidx, oidx, n1idx, n2idx,
               srows, prows, orows, e1rows, e2rows,
               tv, f1v, f2v, sem_p, sem_1, sem_2):
    wid = lax.axis_index("s") * 2 + lax.axis_index("c")

    pltpu.sync_copy(fs_hbm.at[pl.ds(wid * PC, PC)], sidx)
    pltpu.sync_copy(fp_hbm.at[pl.ds(wid * PC, PC)], pidx)
    pltpu.sync_copy(fo_hbm.at[pl.ds(wid * PC, PC)], oidx)
    pltpu.sync_copy(no1_hbm.at[pl.ds(wid * NC, NC)], n1idx)
    pltpu.sync_copy(ns2_hbm.at[pl.ds(wid * NC, NC)], n2idx)

    pos_h = []
    for c in range(PC):
        dst = pl.ds(c * 128, 128)
        pos_h.append(pltpu.async_copy(etab.at[sidx.at[c]], srows.at[dst], sem_p))
        pos_h.append(pltpu.async_copy(rtab.at[pidx.at[c]], prows.at[dst], sem_p))
        pos_h.append(pltpu.async_copy(etab.at[oidx.at[c]], orows.at[dst], sem_p))
    n1_h = [pltpu.async_copy(etab.at[n1idx.at[c]], e1rows.at[pl.ds(c * 128, 128)],
                             sem_1) for c in range(NC)]
    n2_h = [pltpu.async_copy(etab.at[n2idx.at[c]], e2rows.at[pl.ds(c * 128, 128)],
                             sem_2) for c in range(NC)]

    iot = lax.iota(jnp.int32, G16)
    cols = [jnp.full((G16,), d, jnp.int32) for d in range(E_DIM)]

    for h in pos_h:
        h.wait()

    def pos_body(g, carry):
        rows = g * G16 + iot
        acc = jnp.zeros((G16,), jnp.float32)
        for d in range(E_DIM):
            sv = plsc.load_gather(srows, [rows, cols[d]])
            pv = plsc.load_gather(prows, [rows, cols[d]])
            ov = plsc.load_gather(orows, [rows, cols[d]])
            acc = acc + sv * pv * ov
        tv[pl.ds(g * G16, G16)] = acc
        return carry

    lax.fori_loop(0, P // G16, pos_body, 0)

    def neg_loop(arows, brows, erows, fv):
        def body(g, carry):
            j = g * G16 + iot
            pr = lax.shift_right_logical(j, 1)
            acc = jnp.zeros((G16,), jnp.float32)
            for d in range(E_DIM):
                av = plsc.load_gather(arows, [pr, cols[d]])
                bv = plsc.load_gather(brows, [pr, cols[d]])
                ev = plsc.load_gather(erows, [j, cols[d]])
                acc = acc + av * bv * ev
            fv[pl.ds(g * G16, G16)] = acc
            return carry
        lax.fori_loop(0, N // G16, body, 0)

    for h in n1_h:
        h.wait()
    neg_loop(srows, prows, e1rows, f1v)
    for h in n2_h:
        h.wait()
    neg_loop(prows, orows, e2rows, f2v)

    pltpu.sync_copy(tv, true_hbm.at[pl.ds(wid * P, P)])
    pltpu.sync_copy(f1v, f1_hbm.at[pl.ds(wid * N, N)])
    pltpu.sync_copy(f2v, f2_hbm.at[pl.ds(wid * N, N)])


_sc_call = pl.kernel(
    _sc_scores,
    out_type=[
        jax.ShapeDtypeStruct((BATCH,), jnp.float32),
        jax.ShapeDtypeStruct((NEGB,), jnp.float32),
        jax.ShapeDtypeStruct((NEGB,), jnp.float32),
    ],
    mesh=_mesh,
    compiler_params=pltpu.CompilerParams(needs_layout_passes=False,
                                         use_tc_tiling_on_sc=False),
    scratch_types=[
        pltpu.VMEM((PC, 128), jnp.int32),
        pltpu.VMEM((PC, 128), jnp.int32),
        pltpu.VMEM((PC, 128), jnp.int32),
        pltpu.VMEM((NC, 128), jnp.int32),
        pltpu.VMEM((NC, 128), jnp.int32),
        pltpu.VMEM((P, E_DIM), jnp.float32),
        pltpu.VMEM((P, E_DIM), jnp.float32),
        pltpu.VMEM((P, E_DIM), jnp.float32),
        pltpu.VMEM((N, E_DIM), jnp.float32),
        pltpu.VMEM((N, E_DIM), jnp.float32),
        pltpu.VMEM((P,), jnp.float32),
        pltpu.VMEM((N,), jnp.float32),
        pltpu.VMEM((N,), jnp.float32),
        pltpu.SemaphoreType.DMA,
        pltpu.SemaphoreType.DMA,
        pltpu.SemaphoreType.DMA,
    ],
)


def _loss_body(t_ref, f1_ref, f2_ref, out_ref):
    def sp_sum(x):
        return jnp.sum(jnp.log1p(jnp.exp(-jnp.abs(x))) + jnp.maximum(x, 0.0))

    a = sp_sum(-t_ref[...]) / BATCH
    b1 = sp_sum(f1_ref[...]) / NEGB
    b2 = sp_sum(f2_ref[...]) / NEGB
    out_ref[0, 0] = a * 0.5 + (b1 + b2) * 0.25


_loss_call = pl.pallas_call(
    _loss_body,
    out_shape=jax.ShapeDtypeStruct((1, 1), jnp.float32),
    out_specs=pl.BlockSpec(memory_space=pltpu.SMEM),
)


def kernel(inputs, neg_o1, neg_s2, entity_table, relation_table):
    fs = inputs[:, 0].reshape(BATCH // 128, 128)
    fp = inputs[:, 1].reshape(BATCH // 128, 128)
    fo = inputs[:, 2].reshape(BATCH // 128, 128)
    no1 = neg_o1.reshape(NEGB // 128, 128)
    ns2 = neg_s2.reshape(NEGB // 128, 128)
    true_score, f1, f2 = _sc_call(fs, fp, fo, no1, ns2,
                                  entity_table, relation_table)
    loss = _loss_call(true_score.reshape(128, 128),
                      f1.reshape(256, 128), f2.reshape(256, 128))
    return (true_score, loss[0, 0])


# speedup vs baseline: 1.0898x; 1.0898x over previous
import jax
import jax.numpy as jnp
from jax import lax
from jax.experimental import pallas as pl
from jax.experimental.pallas import tpu as pltpu
from jax.experimental.pallas import tpu_sc as plsc

E_DIM = 32
BATCH = 16384
NEGB = 32768
NW = 32
P = BATCH // NW
N = NEGB // NW
PC = P // 128
NC = N // 128

_mesh = plsc.VectorSubcoreMesh(core_axis_name="c", subcore_axis_name="s")


def _sc_gather(fs_hbm, fp_hbm, fo_hbm, no1_hbm, ns2_hbm, etab, rtab,
               s_hbm, p_hbm, o_hbm, e1_hbm, e2_hbm,
               sidx, pidx, oidx, n1idx, n2idx,
               srows, prows, orows, e1rows, e2rows,
               sem_p, sem_1, sem_2, sem_o):
    wid = lax.axis_index("s") * 2 + lax.axis_index("c")

    pltpu.sync_copy(fs_hbm.at[pl.ds(wid * PC, PC)], sidx)
    pltpu.sync_copy(fp_hbm.at[pl.ds(wid * PC, PC)], pidx)
    pltpu.sync_copy(fo_hbm.at[pl.ds(wid * PC, PC)], oidx)
    pltpu.sync_copy(no1_hbm.at[pl.ds(wid * NC, NC)], n1idx)
    pltpu.sync_copy(ns2_hbm.at[pl.ds(wid * NC, NC)], n2idx)

    pos_h = []
    for c in range(PC):
        dst = pl.ds(c * 128, 128)
        pos_h.append(pltpu.async_copy(etab.at[sidx.at[c]], srows.at[dst], sem_p))
        pos_h.append(pltpu.async_copy(rtab.at[pidx.at[c]], prows.at[dst], sem_p))
        pos_h.append(pltpu.async_copy(etab.at[oidx.at[c]], orows.at[dst], sem_p))
    n1_h = [pltpu.async_copy(etab.at[n1idx.at[c]], e1rows.at[pl.ds(c * 128, 128)],
                             sem_1) for c in range(NC)]
    n2_h = [pltpu.async_copy(etab.at[n2idx.at[c]], e2rows.at[pl.ds(c * 128, 128)],
                             sem_2) for c in range(NC)]

    for h in pos_h:
        h.wait()
    out_h = [
        pltpu.async_copy(srows, s_hbm.at[pl.ds(wid * P, P)], sem_o),
        pltpu.async_copy(prows, p_hbm.at[pl.ds(wid * P, P)], sem_o),
        pltpu.async_copy(orows, o_hbm.at[pl.ds(wid * P, P)], sem_o),
    ]
    for h in n1_h:
        h.wait()
    out_h.append(pltpu.async_copy(e1rows, e1_hbm.at[pl.ds(wid * N, N)], sem_o))
    for h in n2_h:
        h.wait()
    out_h.append(pltpu.async_copy(e2rows, e2_hbm.at[pl.ds(wid * N, N)], sem_o))
    for h in out_h:
        h.wait()


_sc_call = pl.kernel(
    _sc_gather,
    out_type=[
        jax.ShapeDtypeStruct((BATCH, E_DIM), jnp.float32),
        jax.ShapeDtypeStruct((BATCH, E_DIM), jnp.float32),
        jax.ShapeDtypeStruct((BATCH, E_DIM), jnp.float32),
        jax.ShapeDtypeStruct((NEGB, E_DIM), jnp.float32),
        jax.ShapeDtypeStruct((NEGB, E_DIM), jnp.float32),
    ],
    mesh=_mesh,
    compiler_params=pltpu.CompilerParams(needs_layout_passes=False,
                                         use_tc_tiling_on_sc=False),
    scratch_types=[
        pltpu.VMEM((PC, 128), jnp.int32),
        pltpu.VMEM((PC, 128), jnp.int32),
        pltpu.VMEM((PC, 128), jnp.int32),
        pltpu.VMEM((NC, 128), jnp.int32),
        pltpu.VMEM((NC, 128), jnp.int32),
        pltpu.VMEM((P, E_DIM), jnp.float32),
        pltpu.VMEM((P, E_DIM), jnp.float32),
        pltpu.VMEM((P, E_DIM), jnp.float32),
        pltpu.VMEM((N, E_DIM), jnp.float32),
        pltpu.VMEM((N, E_DIM), jnp.float32),
        pltpu.SemaphoreType.DMA,
        pltpu.SemaphoreType.DMA,
        pltpu.SemaphoreType.DMA,
        pltpu.SemaphoreType.DMA,
    ],
)


GRID = 8
BP = BATCH // GRID


def _softplus_sum(x):
    return jnp.sum(jnp.log1p(jnp.exp(-jnp.abs(x))) + jnp.maximum(x, 0.0))


def _tc_body(s_ref, p_ref, o_ref, e1_ref, e2_ref, true_ref, loss_ref):
    i = pl.program_id(0)
    s = s_ref[...]
    p = p_ref[...]
    o = o_ref[...]
    sp = s * p
    po = p * o
    spo = sp * o

    t = jnp.sum(spo, axis=1, keepdims=True)
    true_ref[...] = t

    seg = (lax.broadcasted_iota(jnp.int32, (2 * E_DIM, 2), 0) // E_DIM ==
           lax.broadcasted_iota(jnp.int32, (2 * E_DIM, 2), 1)).astype(jnp.float32)

    spc = jnp.concatenate([sp, sp], axis=1)
    poc = jnp.concatenate([po, po], axis=1)
    f1 = lax.dot_general(spc * e1_ref[...], seg, (((1,), (0,)), ((), ())),
                         preferred_element_type=jnp.float32)
    f2 = lax.dot_general(poc * e2_ref[...], seg, (((1,), (0,)), ((), ())),
                         preferred_element_type=jnp.float32)

    part = (_softplus_sum(-t) * (0.5 / BATCH) +
            (_softplus_sum(f1) + _softplus_sum(f2)) * (0.25 / NEGB))

    @pl.when(i == 0)
    def _():
        loss_ref[0, 0] = 0.0

    loss_ref[0, 0] += part


_tc_call = pl.pallas_call(
    _tc_body,
    grid=(GRID,),
    in_specs=[
        pl.BlockSpec((BP, E_DIM), lambda i: (i, 0)),
        pl.BlockSpec((BP, E_DIM), lambda i: (i, 0)),
        pl.BlockSpec((BP, E_DIM), lambda i: (i, 0)),
        pl.BlockSpec((BP, 2 * E_DIM), lambda i: (i, 0)),
        pl.BlockSpec((BP, 2 * E_DIM), lambda i: (i, 0)),
    ],
    out_specs=[
        pl.BlockSpec((BP, 1), lambda i: (i, 0)),
        pl.BlockSpec(memory_space=pltpu.SMEM, index_map=lambda i: (0, 0)),
    ],
    out_shape=[
        jax.ShapeDtypeStruct((BATCH, 1), jnp.float32),
        jax.ShapeDtypeStruct((1, 1), jnp.float32),
    ],
)


def kernel(inputs, neg_o1, neg_s2, entity_table, relation_table):
    fs = inputs[:, 0].reshape(BATCH // 128, 128)
    fp = inputs[:, 1].reshape(BATCH // 128, 128)
    fo = inputs[:, 2].reshape(BATCH // 128, 128)
    no1 = neg_o1.reshape(NEGB // 128, 128)
    ns2 = neg_s2.reshape(NEGB // 128, 128)
    s, p, o, e1, e2 = _sc_call(fs, fp, fo, no1, ns2,
                               entity_table, relation_table)
    true2d, loss = _tc_call(s, p, o,
                            e1.reshape(BATCH, 2 * E_DIM),
                            e2.reshape(BATCH, 2 * E_DIM))
    return (true2d.reshape(BATCH), loss[0, 0])


# speedup vs baseline: 1.1633x; 1.0674x over previous
import jax
import jax.numpy as jnp
from jax import lax
from jax.experimental import pallas as pl
from jax.experimental.pallas import tpu as pltpu
from jax.experimental.pallas import tpu_sc as plsc

E_DIM = 32
BATCH = 16384
NEGB = 32768
NW = 32
P = BATCH // NW
N = NEGB // NW
PC = P // 128
NC = N // 128

_mesh = plsc.VectorSubcoreMesh(core_axis_name="c", subcore_axis_name="s")


def _sc_gather(fs_hbm, fp_hbm, fo_hbm, no1_hbm, ns2_hbm, etab, rtab,
               s_hbm, p_hbm, o_hbm, e1_hbm, e2_hbm,
               sidx, pidx, oidx, n1idx, n2idx,
               srows, prows, orows, e1rows, e2rows,
               sem_p, sem_1, sem_2, sem_o):
    wid = lax.axis_index("s") * 2 + lax.axis_index("c")

    pltpu.sync_copy(fs_hbm.at[pl.ds(wid * PC, PC)], sidx)
    pltpu.sync_copy(fp_hbm.at[pl.ds(wid * PC, PC)], pidx)
    pltpu.sync_copy(fo_hbm.at[pl.ds(wid * PC, PC)], oidx)
    pltpu.sync_copy(no1_hbm.at[pl.ds(wid * NC, NC)], n1idx)
    pltpu.sync_copy(ns2_hbm.at[pl.ds(wid * NC, NC)], n2idx)

    pos_h = []
    for c in range(PC):
        dst = pl.ds(c * 128, 128)
        pos_h.append(pltpu.async_copy(etab.at[sidx.at[c]], srows.at[dst], sem_p))
        pos_h.append(pltpu.async_copy(rtab.at[pidx.at[c]], prows.at[dst], sem_p))
        pos_h.append(pltpu.async_copy(etab.at[oidx.at[c]], orows.at[dst], sem_p))
    n1_h = [pltpu.async_copy(etab.at[n1idx.at[c]], e1rows.at[pl.ds(c * 128, 128)],
                             sem_1) for c in range(NC)]
    n2_h = [pltpu.async_copy(etab.at[n2idx.at[c]], e2rows.at[pl.ds(c * 128, 128)],
                             sem_2) for c in range(NC)]

    for h in pos_h:
        h.wait()
    out_h = [
        pltpu.async_copy(srows, s_hbm.at[pl.ds(wid * P, P)], sem_o),
        pltpu.async_copy(prows, p_hbm.at[pl.ds(wid * P, P)], sem_o),
        pltpu.async_copy(orows, o_hbm.at[pl.ds(wid * P, P)], sem_o),
    ]
    for h in n1_h:
        h.wait()
    out_h.append(pltpu.async_copy(e1rows, e1_hbm.at[pl.ds(wid * N, N)], sem_o))
    for h in n2_h:
        h.wait()
    out_h.append(pltpu.async_copy(e2rows, e2_hbm.at[pl.ds(wid * N, N)], sem_o))
    for h in out_h:
        h.wait()


_sc_call = pl.kernel(
    _sc_gather,
    out_type=[
        jax.ShapeDtypeStruct((BATCH, E_DIM), jnp.float32),
        jax.ShapeDtypeStruct((BATCH, E_DIM), jnp.float32),
        jax.ShapeDtypeStruct((BATCH, E_DIM), jnp.float32),
        jax.ShapeDtypeStruct((NEGB, E_DIM), jnp.float32),
        jax.ShapeDtypeStruct((NEGB, E_DIM), jnp.float32),
    ],
    mesh=_mesh,
    compiler_params=pltpu.CompilerParams(needs_layout_passes=False,
                                         use_tc_tiling_on_sc=False),
    scratch_types=[
        pltpu.VMEM((PC, 128), jnp.int32),
        pltpu.VMEM((PC, 128), jnp.int32),
        pltpu.VMEM((PC, 128), jnp.int32),
        pltpu.VMEM((NC, 128), jnp.int32),
        pltpu.VMEM((NC, 128), jnp.int32),
        pltpu.VMEM((P, E_DIM), jnp.float32),
        pltpu.VMEM((P, E_DIM), jnp.float32),
        pltpu.VMEM((P, E_DIM), jnp.float32),
        pltpu.VMEM((N, E_DIM), jnp.float32),
        pltpu.VMEM((N, E_DIM), jnp.float32),
        pltpu.SemaphoreType.DMA,
        pltpu.SemaphoreType.DMA,
        pltpu.SemaphoreType.DMA,
        pltpu.SemaphoreType.DMA,
    ],
)


GRID = 8
RPOS = BATCH * E_DIM // 128
BPR = RPOS // GRID
PACK = 128 // E_DIM


def _softplus_sum(x):
    return jnp.sum(jnp.log1p(jnp.exp(-jnp.abs(x))) + jnp.maximum(x, 0.0))


def _tc_body(s_ref, p_ref, o_ref, e1e_ref, e1o_ref, e2e_ref, e2o_ref,
             true_ref, loss_ref):
    i = pl.program_id(0)
    sp = s_ref[...] * p_ref[...]
    po = p_ref[...] * o_ref[...]
    spo = sp * o_ref[...]

    seg = (lax.broadcasted_iota(jnp.int32, (128, PACK), 0) // E_DIM ==
           lax.broadcasted_iota(jnp.int32, (128, PACK), 1)).astype(jnp.float32)

    def segdot(x):
        return lax.dot_general(x, seg, (((1,), (0,)), ((), ())),
                               preferred_element_type=jnp.float32)

    t = segdot(spo)
    true_ref[...] = t
    f1 = _softplus_sum(segdot(sp * e1e_ref[...])) + \
        _softplus_sum(segdot(sp * e1o_ref[...]))
    f2 = _softplus_sum(segdot(po * e2e_ref[...])) + \
        _softplus_sum(segdot(po * e2o_ref[...]))

    part = _softplus_sum(-t) * (0.5 / BATCH) + (f1 + f2) * (0.25 / NEGB)

    @pl.when(i == 0)
    def _():
        loss_ref[0, 0] = 0.0

    loss_ref[0, 0] += part


_tc_call = pl.pallas_call(
    _tc_body,
    grid=(GRID,),
    in_specs=[
        pl.BlockSpec((BPR, 128), lambda i: (i, 0)),
        pl.BlockSpec((BPR, 128), lambda i: (i, 0)),
        pl.BlockSpec((BPR, 128), lambda i: (i, 0)),
        pl.BlockSpec((BPR, 128), lambda i: (i, 0)),
        pl.BlockSpec((BPR, 128), lambda i: (i + GRID, 0)),
        pl.BlockSpec((BPR, 128), lambda i: (i, 0)),
        pl.BlockSpec((BPR, 128), lambda i: (i + GRID, 0)),
    ],
    out_specs=[
        pl.BlockSpec((BPR, PACK), lambda i: (i, 0)),
        pl.BlockSpec(memory_space=pltpu.SMEM, index_map=lambda i: (0, 0)),
    ],
    out_shape=[
        jax.ShapeDtypeStruct((RPOS, PACK), jnp.float32),
        jax.ShapeDtypeStruct((1, 1), jnp.float32),
    ],
)


def kernel(inputs, neg_o1, neg_s2, entity_table, relation_table):
    fs = inputs[:, 0].reshape(BATCH // 128, 128)
    fp = inputs[:, 1].reshape(BATCH // 128, 128)
    fo = inputs[:, 2].reshape(BATCH // 128, 128)
    no1 = neg_o1.reshape(BATCH, 2).T.reshape(NEGB // 128, 128)
    ns2 = neg_s2.reshape(BATCH, 2).T.reshape(NEGB // 128, 128)
    s, p, o, e1, e2 = _sc_call(fs, fp, fo, no1, ns2,
                               entity_table, relation_table)
    sr = s.reshape(RPOS, 128)
    pr = p.reshape(RPOS, 128)
    orr = o.reshape(RPOS, 128)
    e1r = e1.reshape(2 * RPOS, 128)
    e2r = e2.reshape(2 * RPOS, 128)
    t4, loss = _tc_call(sr, pr, orr, e1r, e1r, e2r, e2r)
    return (t4.reshape(BATCH), loss[0, 0])


# speedup vs baseline: 1.3738x; 1.1809x over previous
import jax
import jax.numpy as jnp
from jax import lax
from jax.experimental import pallas as pl
from jax.experimental.pallas import tpu as pltpu
from jax.experimental.pallas import tpu_sc as plsc

E_DIM = 32
BATCH = 16384
NEGB = 32768
NENT = 1000000
NW = 32
P = BATCH // NW
N = NEGB // NW
PC = P // 128
NC = N // 128

PACK_C = 8192
PACK_G = -(-NENT // PACK_C)
PROWS = PACK_G * 2048


def _pack_body(a0, a1, a2, a3, out_ref):
    out_ref[...] = jnp.concatenate(
        [a0[...].T, a1[...].T, a2[...].T, a3[...].T], axis=1)


_FULL = NENT // 2048 - 1


def _pack_idx(q):
    if q == 0:
        return lambda i: (0, jnp.minimum(4 * i, _FULL + 1))
    return lambda i: (0, jnp.minimum(4 * i + q, _FULL))


_pack_call = pl.pallas_call(
    _pack_body,
    grid=(PACK_G,),
    in_specs=[pl.BlockSpec((E_DIM, 2048), _pack_idx(q)) for q in range(4)],
    out_specs=pl.BlockSpec((2048, 128), lambda i: (i, 0)),
    out_shape=jax.ShapeDtypeStruct((PROWS, 128), jnp.float32),
)

_mesh = plsc.VectorSubcoreMesh(core_axis_name="c", subcore_axis_name="s")


def _rowoff(idx_ref, row_ref, off_ref, nrows):
    def body(k, carry):
        r = k // 8
        g = k % 8
        v = idx_ref[r, pl.ds(g * 16, 16)]
        u = v & (PACK_C - 1)
        row_ref[r, pl.ds(g * 16, 16)] = ((v >> 13) << 11) + (u & 2047)
        off_ref[r, pl.ds(g * 16, 16)] = (u >> 11) * E_DIM
        return carry
    lax.fori_loop(0, nrows * 8, body, jnp.int32(0))


def _sc_body(fs_hbm, fp_hbm, fo_hbm, n1_hbm, n2_hbm, pt, rtab,
             t_hbm, f1_hbm, f2_hbm,
             sidx, pidx, oidx, n1idx, n2idx,
             srow, soff, orow, ooff, n1row, n1off, n2row, n2off,
             spk, opk, prow, spc, poc, epk,
             tsc, f1sc, f2sc,
             sem_g, sem_o):
    wid = lax.axis_index("s") * 2 + lax.axis_index("c")
    iota = lax.iota(jnp.int32, 16)

    pltpu.sync_copy(fs_hbm.at[pl.ds(wid * PC, PC)], sidx)
    pltpu.sync_copy(fp_hbm.at[pl.ds(wid * PC, PC)], pidx)
    pltpu.sync_copy(fo_hbm.at[pl.ds(wid * PC, PC)], oidx)
    pltpu.sync_copy(n1_hbm.at[pl.ds(wid * PC, PC)], n1idx.at[pl.ds(0, PC)])
    pltpu.sync_copy(n1_hbm.at[pl.ds(128 + wid * PC, PC)],
                    n1idx.at[pl.ds(PC, PC)])
    pltpu.sync_copy(n2_hbm.at[pl.ds(wid * PC, PC)], n2idx.at[pl.ds(0, PC)])
    pltpu.sync_copy(n2_hbm.at[pl.ds(128 + wid * PC, PC)],
                    n2idx.at[pl.ds(PC, PC)])

    _rowoff(sidx, srow, soff, PC)
    _rowoff(oidx, orow, ooff, PC)
    _rowoff(n1idx, n1row, n1off, NC)
    _rowoff(n2idx, n2row, n2off, NC)

    def pos_chunk(c, carry):
        hs = pltpu.async_copy(pt.at[srow.at[c]], spk, sem_g)
        ho = pltpu.async_copy(pt.at[orow.at[c]], opk, sem_g)
        hp = pltpu.async_copy(rtab.at[pidx.at[c]], prow, sem_g)
        hs.wait()
        ho.wait()
        hp.wait()

        def grp(g, carry2):
            jl = g * 16 + iota
            so = soff[c, pl.ds(g * 16, 16)]
            oo = ooff[c, pl.ds(g * 16, 16)]
            acc = jnp.zeros((16,), jnp.float32)
            for e in range(E_DIM):
                ec = jnp.full((16,), e, jnp.int32)
                sv = plsc.load_gather(spk, [jl, so + e])
                ov = plsc.load_gather(opk, [jl, oo + e])
                pv = plsc.load_gather(prow, [jl, ec])
                sp = sv * pv
                po = pv * ov
                acc = acc + sp * ov
                plsc.store_scatter(spc, [c * 128 + jl, ec], sp)
                plsc.store_scatter(poc, [c * 128 + jl, ec], po)
            tsc[c, pl.ds(g * 16, 16)] = acc
            return carry2
        lax.fori_loop(0, 8, grp, jnp.int32(0))
        return carry
    lax.fori_loop(0, PC, pos_chunk, jnp.int32(0))

    def neg_pass(row_ref, off_ref, qc_ref, out_ref):
        def neg_chunk(c, carry):
            h = pltpu.async_copy(pt.at[row_ref.at[c]], epk, sem_g)
            h.wait()

            def grp(g, carry2):
                jl = g * 16 + iota
                eo = off_ref[c, pl.ds(g * 16, 16)]
                qrow = (c % PC) * 128 + jl
                acc = jnp.zeros((16,), jnp.float32)
                for e in range(E_DIM):
                    ec = jnp.full((16,), e, jnp.int32)
                    ev = plsc.load_gather(epk, [jl, eo + e])
                    qv = plsc.load_gather(qc_ref, [qrow, ec])
                    acc = acc + ev * qv
                out_ref[c, pl.ds(g * 16, 16)] = acc
                return carry2
            lax.fori_loop(0, 8, grp, jnp.int32(0))
            return carry
        lax.fori_loop(0, NC, neg_chunk, jnp.int32(0))

    neg_pass(n1row, n1off, spc, f1sc)
    neg_pass(n2row, n2off, poc, f2sc)

    hs = [
        pltpu.async_copy(tsc, t_hbm.at[pl.ds(wid * PC, PC)], sem_o),
        pltpu.async_copy(f1sc.at[pl.ds(0, PC)],
                         f1_hbm.at[pl.ds(wid * PC, PC)], sem_o),
        pltpu.async_copy(f1sc.at[pl.ds(PC, PC)],
                         f1_hbm.at[pl.ds(128 + wid * PC, PC)], sem_o),
        pltpu.async_copy(f2sc.at[pl.ds(0, PC)],
                         f2_hbm.at[pl.ds(wid * PC, PC)], sem_o),
        pltpu.async_copy(f2sc.at[pl.ds(PC, PC)],
                         f2_hbm.at[pl.ds(128 + wid * PC, PC)], sem_o),
    ]
    for h in hs:
        h.wait()


_sc_call = pl.kernel(
    _sc_body,
    out_type=[
        jax.ShapeDtypeStruct((BATCH // 128, 128), jnp.float32),
        jax.ShapeDtypeStruct((NEGB // 128, 128), jnp.float32),
        jax.ShapeDtypeStruct((NEGB // 128, 128), jnp.float32),
    ],
    mesh=_mesh,
    compiler_params=pltpu.CompilerParams(needs_layout_passes=False,
                                         use_tc_tiling_on_sc=False),
    scratch_types=[
        pltpu.VMEM((PC, 128), jnp.int32),
        pltpu.VMEM((PC, 128), jnp.int32),
        pltpu.VMEM((PC, 128), jnp.int32),
        pltpu.VMEM((NC, 128), jnp.int32),
        pltpu.VMEM((NC, 128), jnp.int32),
        pltpu.VMEM((PC, 128), jnp.int32),
        pltpu.VMEM((PC, 128), jnp.int32),
        pltpu.VMEM((PC, 128), jnp.int32),
        pltpu.VMEM((PC, 128), jnp.int32),
        pltpu.VMEM((NC, 128), jnp.int32),
        pltpu.VMEM((NC, 128), jnp.int32),
        pltpu.VMEM((NC, 128), jnp.int32),
        pltpu.VMEM((NC, 128), jnp.int32),
        pltpu.VMEM((128, 128), jnp.float32),
        pltpu.VMEM((128, 128), jnp.float32),
        pltpu.VMEM((128, E_DIM), jnp.float32),
        pltpu.VMEM((P, E_DIM), jnp.float32),
        pltpu.VMEM((P, E_DIM), jnp.float32),
        pltpu.VMEM((128, 128), jnp.float32),
        pltpu.VMEM((PC, 128), jnp.float32),
        pltpu.VMEM((NC, 128), jnp.float32),
        pltpu.VMEM((NC, 128), jnp.float32),
        pltpu.SemaphoreType.DMA,
        pltpu.SemaphoreType.DMA,
    ],
)


def _softplus_sum(x):
    return jnp.sum(jnp.log1p(jnp.exp(-jnp.abs(x))) + jnp.maximum(x, 0.0))


def _loss_body(t_ref, f1_ref, f2_ref, loss_ref):
    t = t_ref[...]
    loss_ref[0, 0] = (_softplus_sum(-t) * (0.5 / BATCH)
                      + (_softplus_sum(f1_ref[...])
                         + _softplus_sum(f2_ref[...])) * (0.25 / NEGB))


_loss_call = pl.pallas_call(
    _loss_body,
    out_specs=pl.BlockSpec(memory_space=pltpu.SMEM),
    out_shape=jax.ShapeDtypeStruct((1, 1), jnp.float32),
)


def kernel(inputs, neg_o1, neg_s2, entity_table, relation_table):
    fs = inputs[:, 0].reshape(BATCH // 128, 128)
    fp = inputs[:, 1].reshape(BATCH // 128, 128)
    fo = inputs[:, 2].reshape(BATCH // 128, 128)
    no1 = neg_o1.reshape(BATCH, 2).T.reshape(NEGB // 128, 128)
    ns2 = neg_s2.reshape(BATCH, 2).T.reshape(NEGB // 128, 128)
    et = entity_table.T
    pt = _pack_call(et, et, et, et)
    t, f1, f2 = _sc_call(fs, fp, fo, no1, ns2, pt, relation_table)
    loss = _loss_call(t, f1, f2)
    return (t.reshape(BATCH), loss[0, 0])


# speedup vs baseline: 1.5333x; 1.1161x over previous
import jax
import jax.numpy as jnp
from jax import lax
from jax.experimental import pallas as pl
from jax.experimental.pallas import tpu as pltpu
from jax.experimental.pallas import tpu_sc as plsc

E_DIM = 32
BATCH = 16384
NEGB = 32768
NENT = 1000000
NW = 32
P = BATCH // NW
N = NEGB // NW
PC = P // 128
NC = N // 128

PACK_C = 8192
PACK_G = -(-NENT // PACK_C)
PROWS = PACK_G * 2048


def _pack_body(a0, a1, a2, a3, out_ref):
    out_ref[...] = jnp.concatenate(
        [a0[...].T, a1[...].T, a2[...].T, a3[...].T], axis=1)


_FULL = NENT // 2048 - 1


def _pack_idx(q):
    if q == 0:
        return lambda i: (0, jnp.minimum(4 * i, _FULL + 1))
    return lambda i: (0, jnp.minimum(4 * i + q, _FULL))


_pack_call = pl.pallas_call(
    _pack_body,
    grid=(PACK_G,),
    in_specs=[pl.BlockSpec((E_DIM, 2048), _pack_idx(q)) for q in range(4)],
    out_specs=pl.BlockSpec((2048, 128), lambda i: (i, 0)),
    out_shape=jax.ShapeDtypeStruct((PROWS, 128), jnp.float32),
)

_mesh = plsc.VectorSubcoreMesh(core_axis_name="c", subcore_axis_name="s")


def _rowoff(idx_ref, row_ref, off_ref, nrows):
    def body(k, carry):
        r = k // 8
        g = k % 8
        v = idx_ref[r, pl.ds(g * 16, 16)]
        u = v & (PACK_C - 1)
        row_ref[r, pl.ds(g * 16, 16)] = ((v >> 13) << 11) + (u & 2047)
        off_ref[r, pl.ds(g * 16, 16)] = (u >> 11) * E_DIM
        return carry
    lax.fori_loop(0, nrows * 8, body, jnp.int32(0))


def _sc_body(fs_hbm, fp_hbm, fo_hbm, n1_hbm, n2_hbm, pt, rtab,
             t_hbm, f1_hbm, f2_hbm,
             sidx, pidx, oidx, n1idx, n2idx,
             srow, soff, orow, ooff, n1row, n1off, n2row, n2off,
             spk, opk, prow, spc, poc, epk,
             tsc, f1sc, f2sc,
             sem_g, sem_o):
    wid = lax.axis_index("s") * 2 + lax.axis_index("c")
    iota = lax.iota(jnp.int32, 16)

    pltpu.sync_copy(fs_hbm.at[pl.ds(wid * PC, PC)], sidx)
    pltpu.sync_copy(fp_hbm.at[pl.ds(wid * PC, PC)], pidx)
    pltpu.sync_copy(fo_hbm.at[pl.ds(wid * PC, PC)], oidx)
    pltpu.sync_copy(n1_hbm.at[pl.ds(wid * PC, PC)], n1idx.at[pl.ds(0, PC)])
    pltpu.sync_copy(n1_hbm.at[pl.ds(128 + wid * PC, PC)],
                    n1idx.at[pl.ds(PC, PC)])
    pltpu.sync_copy(n2_hbm.at[pl.ds(wid * PC, PC)], n2idx.at[pl.ds(0, PC)])
    pltpu.sync_copy(n2_hbm.at[pl.ds(128 + wid * PC, PC)],
                    n2idx.at[pl.ds(PC, PC)])

    _rowoff(sidx, srow, soff, PC)
    _rowoff(oidx, orow, ooff, PC)
    _rowoff(n1idx, n1row, n1off, NC)
    _rowoff(n2idx, n2row, n2off, NC)

    def pos_chunk(c, carry):
        hs = pltpu.async_copy(pt.at[srow.at[c]], spk, sem_g)
        ho = pltpu.async_copy(pt.at[orow.at[c]], opk, sem_g)
        hp = pltpu.async_copy(rtab.at[pidx.at[c]], prow, sem_g)
        hs.wait()
        ho.wait()
        hp.wait()

        def grp(g, carry2):
            jl = g * 16 + iota
            base = c * 128 + g * 16
            so = soff[c, pl.ds(g * 16, 16)]
            oo = ooff[c, pl.ds(g * 16, 16)]
            acc = jnp.zeros((16,), jnp.float32)
            for e in range(E_DIM):
                ec = jnp.full((16,), e, jnp.int32)
                sv = plsc.load_gather(spk, [jl, so + e])
                ov = plsc.load_gather(opk, [jl, oo + e])
                pv = plsc.load_gather(prow, [jl, ec])
                sp = sv * pv
                po = pv * ov
                acc = acc + sp * ov
                spc[e, pl.ds(base, 16)] = sp
                poc[e, pl.ds(base, 16)] = po
            tsc[c, pl.ds(g * 16, 16)] = acc
            return carry2
        lax.fori_loop(0, 8, grp, jnp.int32(0))
        return carry
    lax.fori_loop(0, PC, pos_chunk, jnp.int32(0))

    def neg_pass(row_ref, off_ref, qc_ref, out_ref):
        def neg_chunk(c, carry):
            h = pltpu.async_copy(pt.at[row_ref.at[c]], epk, sem_g)
            h.wait()

            def grp(g, carry2):
                jl = g * 16 + iota
                eo = off_ref[c, pl.ds(g * 16, 16)]
                qbase = (c % PC) * 128 + g * 16
                acc = jnp.zeros((16,), jnp.float32)
                for e in range(E_DIM):
                    ev = plsc.load_gather(epk, [jl, eo + e])
                    acc = acc + ev * qc_ref[e, pl.ds(qbase, 16)]
                out_ref[c, pl.ds(g * 16, 16)] = acc
                return carry2
            lax.fori_loop(0, 8, grp, jnp.int32(0))
            return carry
        lax.fori_loop(0, NC, neg_chunk, jnp.int32(0))

    neg_pass(n1row, n1off, spc, f1sc)
    neg_pass(n2row, n2off, poc, f2sc)

    hs = [
        pltpu.async_copy(tsc, t_hbm.at[pl.ds(wid * PC, PC)], sem_o),
        pltpu.async_copy(f1sc.at[pl.ds(0, PC)],
                         f1_hbm.at[pl.ds(wid * PC, PC)], sem_o),
        pltpu.async_copy(f1sc.at[pl.ds(PC, PC)],
                         f1_hbm.at[pl.ds(128 + wid * PC, PC)], sem_o),
        pltpu.async_copy(f2sc.at[pl.ds(0, PC)],
                         f2_hbm.at[pl.ds(wid * PC, PC)], sem_o),
        pltpu.async_copy(f2sc.at[pl.ds(PC, PC)],
                         f2_hbm.at[pl.ds(128 + wid * PC, PC)], sem_o),
    ]
    for h in hs:
        h.wait()


_sc_call = pl.kernel(
    _sc_body,
    out_type=[
        jax.ShapeDtypeStruct((BATCH // 128, 128), jnp.float32),
        jax.ShapeDtypeStruct((NEGB // 128, 128), jnp.float32),
        jax.ShapeDtypeStruct((NEGB // 128, 128), jnp.float32),
    ],
    mesh=_mesh,
    compiler_params=pltpu.CompilerParams(needs_layout_passes=False,
                                         use_tc_tiling_on_sc=False),
    scratch_types=[
        pltpu.VMEM((PC, 128), jnp.int32),
        pltpu.VMEM((PC, 128), jnp.int32),
        pltpu.VMEM((PC, 128), jnp.int32),
        pltpu.VMEM((NC, 128), jnp.int32),
        pltpu.VMEM((NC, 128), jnp.int32),
        pltpu.VMEM((PC, 128), jnp.int32),
        pltpu.VMEM((PC, 128), jnp.int32),
        pltpu.VMEM((PC, 128), jnp.int32),
        pltpu.VMEM((PC, 128), jnp.int32),
        pltpu.VMEM((NC, 128), jnp.int32),
        pltpu.VMEM((NC, 128), jnp.int32),
        pltpu.VMEM((NC, 128), jnp.int32),
        pltpu.VMEM((NC, 128), jnp.int32),
        pltpu.VMEM((128, 128), jnp.float32),
        pltpu.VMEM((128, 128), jnp.float32),
        pltpu.VMEM((128, E_DIM), jnp.float32),
        pltpu.VMEM((E_DIM, P), jnp.float32),
        pltpu.VMEM((E_DIM, P), jnp.float32),
        pltpu.VMEM((128, 128), jnp.float32),
        pltpu.VMEM((PC, 128), jnp.float32),
        pltpu.VMEM((NC, 128), jnp.float32),
        pltpu.VMEM((NC, 128), jnp.float32),
        pltpu.SemaphoreType.DMA,
        pltpu.SemaphoreType.DMA,
    ],
)


def _softplus_sum(x):
    return jnp.sum(jnp.log1p(jnp.exp(-jnp.abs(x))) + jnp.maximum(x, 0.0))


def _loss_body(t_ref, f1_ref, f2_ref, loss_ref):
    t = t_ref[...]
    loss_ref[0, 0] = (_softplus_sum(-t) * (0.5 / BATCH)
                      + (_softplus_sum(f1_ref[...])
                         + _softplus_sum(f2_ref[...])) * (0.25 / NEGB))


_loss_call = pl.pallas_call(
    _loss_body,
    out_specs=pl.BlockSpec(memory_space=pltpu.SMEM),
    out_shape=jax.ShapeDtypeStruct((1, 1), jnp.float32),
)


def kernel(inputs, neg_o1, neg_s2, entity_table, relation_table):
    fs = inputs[:, 0].reshape(BATCH // 128, 128)
    fp = inputs[:, 1].reshape(BATCH // 128, 128)
    fo = inputs[:, 2].reshape(BATCH // 128, 128)
    no1 = neg_o1.reshape(BATCH, 2).T.reshape(NEGB // 128, 128)
    ns2 = neg_s2.reshape(BATCH, 2).T.reshape(NEGB // 128, 128)
    et = entity_table.T
    pt = _pack_call(et, et, et, et)
    t, f1, f2 = _sc_call(fs, fp, fo, no1, ns2, pt, relation_table)
    loss = _loss_call(t, f1, f2)
    return (t.reshape(BATCH), loss[0, 0])


# speedup vs baseline: 1.5903x; 1.0372x over previous
import jax
import jax.numpy as jnp
from jax import lax
from jax.experimental import pallas as pl
from jax.experimental.pallas import tpu as pltpu
from jax.experimental.pallas import tpu_sc as plsc

E_DIM = 32
BATCH = 16384
NEGB = 32768
NENT = 1000000
NW = 32
P = BATCH // NW
N = NEGB // NW
PC = P // 128
NC = N // 128

PACK_C = 8192
PACK_G = -(-NENT // PACK_C)
PROWS = PACK_G * 2048


def _pack_body(a0, a1, a2, a3, out_ref):
    out_ref[...] = jnp.concatenate(
        [a0[...].T, a1[...].T, a2[...].T, a3[...].T], axis=1)


_FULL = NENT // 2048 - 1


def _pack_idx(q):
    if q == 0:
        return lambda i: (0, jnp.minimum(4 * i, _FULL + 1))
    return lambda i: (0, jnp.minimum(4 * i + q, _FULL))


_pack_call = pl.pallas_call(
    _pack_body,
    grid=(PACK_G,),
    in_specs=[pl.BlockSpec((E_DIM, 2048), _pack_idx(q)) for q in range(4)],
    out_specs=pl.BlockSpec((2048, 128), lambda i: (i, 0)),
    out_shape=jax.ShapeDtypeStruct((PROWS, 128), jnp.float32),
)

_mesh = plsc.VectorSubcoreMesh(core_axis_name="c", subcore_axis_name="s")


def _rowoff(idx_ref, row_ref, off_ref, nrows):
    def body(k, carry):
        r = k // 8
        g = k % 8
        v = idx_ref[r, pl.ds(g * 16, 16)]
        u = v & (PACK_C - 1)
        row_ref[r, pl.ds(g * 16, 16)] = ((v >> 13) << 11) + (u & 2047)
        off_ref[r, pl.ds(g * 16, 16)] = (u >> 11) * E_DIM
        return carry
    lax.fori_loop(0, nrows * 8, body, jnp.int32(0))


def _sc_body(fs_hbm, fp_hbm, fo_hbm, n1_hbm, n2_hbm, pt, rtab,
             t_hbm, f1_hbm, f2_hbm,
             sidx, pidx, oidx, n1idx, n2idx,
             srow, soff, orow, ooff, n1row, n1off, n2row, n2off,
             spk, opk, prow, spc, poc, epk,
             tsc, f1sc, f2sc,
             sem_g, sem_o):
    wid = lax.axis_index("s") * 2 + lax.axis_index("c")
    iota = lax.iota(jnp.int32, 16)

    pltpu.sync_copy(fs_hbm.at[pl.ds(wid * PC, PC)], sidx)
    pltpu.sync_copy(fp_hbm.at[pl.ds(wid * PC, PC)], pidx)
    pltpu.sync_copy(fo_hbm.at[pl.ds(wid * PC, PC)], oidx)
    pltpu.sync_copy(n1_hbm.at[pl.ds(wid * PC, PC)], n1idx.at[pl.ds(0, PC)])
    pltpu.sync_copy(n1_hbm.at[pl.ds(128 + wid * PC, PC)],
                    n1idx.at[pl.ds(PC, PC)])
    pltpu.sync_copy(n2_hbm.at[pl.ds(wid * PC, PC)], n2idx.at[pl.ds(0, PC)])
    pltpu.sync_copy(n2_hbm.at[pl.ds(128 + wid * PC, PC)],
                    n2idx.at[pl.ds(PC, PC)])

    _rowoff(sidx, srow, soff, PC)
    _rowoff(oidx, orow, ooff, PC)
    _rowoff(n1idx, n1row, n1off, NC)
    _rowoff(n2idx, n2row, n2off, NC)

    def pos_chunk(c, carry):
        hs = pltpu.async_copy(pt.at[srow.at[c]], spk, sem_g)
        ho = pltpu.async_copy(pt.at[orow.at[c]], opk, sem_g)
        hp = pltpu.async_copy(rtab.at[pidx.at[c]], prow, sem_g)
        hs.wait()
        ho.wait()
        hp.wait()

        def grp(g, carry2):
            jl = g * 16 + iota
            base = c * 128 + g * 16
            so = soff[c, pl.ds(g * 16, 16)]
            oo = ooff[c, pl.ds(g * 16, 16)]
            acc = jnp.zeros((16,), jnp.float32)
            for e in range(E_DIM):
                ec = jnp.full((16,), e, jnp.int32)
                sv = plsc.load_gather(spk, [jl, so + e])
                ov = plsc.load_gather(opk, [jl, oo + e])
                pv = plsc.load_gather(prow, [jl, ec])
                sp = sv * pv
                po = pv * ov
                acc = acc + sp * ov
                spc[e, pl.ds(base, 16)] = sp
                poc[e, pl.ds(base, 16)] = po
            tsc[c, pl.ds(g * 16, 16)] = acc
            return carry2
        lax.fori_loop(0, 8, grp, jnp.int32(0))
        return carry
    lax.fori_loop(0, PC, pos_chunk, jnp.int32(0))

    def neg_pass(row_ref, off_ref, qc_ref, out_ref):
        bufs = (epk, spk)
        sems = (sem_g, sem_o)
        h = pltpu.async_copy(pt.at[row_ref.at[0]], bufs[0], sems[0])
        for c in range(NC):
            h.wait()
            if c + 1 < NC:
                h = pltpu.async_copy(pt.at[row_ref.at[c + 1]],
                                     bufs[(c + 1) % 2], sems[(c + 1) % 2])
            buf = bufs[c % 2]

            def grp(g, carry2, c=c, buf=buf):
                jl = g * 16 + iota
                eo = off_ref[c, pl.ds(g * 16, 16)]
                qbase = (c % PC) * 128 + g * 16
                acc = jnp.zeros((16,), jnp.float32)
                for e in range(E_DIM):
                    ev = plsc.load_gather(buf, [jl, eo + e])
                    acc = acc + ev * qc_ref[e, pl.ds(qbase, 16)]
                out_ref[c, pl.ds(g * 16, 16)] = acc
                return carry2
            lax.fori_loop(0, 8, grp, jnp.int32(0))

    neg_pass(n1row, n1off, spc, f1sc)
    neg_pass(n2row, n2off, poc, f2sc)

    hs = [
        pltpu.async_copy(tsc, t_hbm.at[pl.ds(wid * PC, PC)], sem_o),
        pltpu.async_copy(f1sc.at[pl.ds(0, PC)],
                         f1_hbm.at[pl.ds(wid * PC, PC)], sem_o),
        pltpu.async_copy(f1sc.at[pl.ds(PC, PC)],
                         f1_hbm.at[pl.ds(128 + wid * PC, PC)], sem_o),
        pltpu.async_copy(f2sc.at[pl.ds(0, PC)],
                         f2_hbm.at[pl.ds(wid * PC, PC)], sem_o),
        pltpu.async_copy(f2sc.at[pl.ds(PC, PC)],
                         f2_hbm.at[pl.ds(128 + wid * PC, PC)], sem_o),
    ]
    for h in hs:
        h.wait()


_sc_call = pl.kernel(
    _sc_body,
    out_type=[
        jax.ShapeDtypeStruct((BATCH // 128, 128), jnp.float32),
        jax.ShapeDtypeStruct((NEGB // 128, 128), jnp.float32),
        jax.ShapeDtypeStruct((NEGB // 128, 128), jnp.float32),
    ],
    mesh=_mesh,
    compiler_params=pltpu.CompilerParams(needs_layout_passes=False,
                                         use_tc_tiling_on_sc=False),
    scratch_types=[
        pltpu.VMEM((PC, 128), jnp.int32),
        pltpu.VMEM((PC, 128), jnp.int32),
        pltpu.VMEM((PC, 128), jnp.int32),
        pltpu.VMEM((NC, 128), jnp.int32),
        pltpu.VMEM((NC, 128), jnp.int32),
        pltpu.VMEM((PC, 128), jnp.int32),
        pltpu.VMEM((PC, 128), jnp.int32),
        pltpu.VMEM((PC, 128), jnp.int32),
        pltpu.VMEM((PC, 128), jnp.int32),
        pltpu.VMEM((NC, 128), jnp.int32),
        pltpu.VMEM((NC, 128), jnp.int32),
        pltpu.VMEM((NC, 128), jnp.int32),
        pltpu.VMEM((NC, 128), jnp.int32),
        pltpu.VMEM((128, 128), jnp.float32),
        pltpu.VMEM((128, 128), jnp.float32),
        pltpu.VMEM((128, E_DIM), jnp.float32),
        pltpu.VMEM((E_DIM, P), jnp.float32),
        pltpu.VMEM((E_DIM, P), jnp.float32),
        pltpu.VMEM((128, 128), jnp.float32),
        pltpu.VMEM((PC, 128), jnp.float32),
        pltpu.VMEM((NC, 128), jnp.float32),
        pltpu.VMEM((NC, 128), jnp.float32),
        pltpu.SemaphoreType.DMA,
        pltpu.SemaphoreType.DMA,
    ],
)


def _softplus_sum(x):
    return jnp.sum(jnp.log1p(jnp.exp(-jnp.abs(x))) + jnp.maximum(x, 0.0))


def _loss_body(t_ref, f1_ref, f2_ref, loss_ref):
    t = t_ref[...]
    loss_ref[0, 0] = (_softplus_sum(-t) * (0.5 / BATCH)
                      + (_softplus_sum(f1_ref[...])
                         + _softplus_sum(f2_ref[...])) * (0.25 / NEGB))


_loss_call = pl.pallas_call(
    _loss_body,
    out_specs=pl.BlockSpec(memory_space=pltpu.SMEM),
    out_shape=jax.ShapeDtypeStruct((1, 1), jnp.float32),
)


def kernel(inputs, neg_o1, neg_s2, entity_table, relation_table):
    fs = inputs[:, 0].reshape(BATCH // 128, 128)
    fp = inputs[:, 1].reshape(BATCH // 128, 128)
    fo = inputs[:, 2].reshape(BATCH // 128, 128)
    no1 = neg_o1.reshape(BATCH, 2).T.reshape(NEGB // 128, 128)
    ns2 = neg_s2.reshape(BATCH, 2).T.reshape(NEGB // 128, 128)
    et = entity_table.T
    pt = _pack_call(et, et, et, et)
    t, f1, f2 = _sc_call(fs, fp, fo, no1, ns2, pt, relation_table)
    loss = _loss_call(t, f1, f2)
    return (t.reshape(BATCH), loss[0, 0])
